# baseline (speedup 1.0000x reference)
import functools

import jax
import jax.numpy as jnp
from jax import lax
from jax.experimental import pallas as pl
from jax.experimental.pallas import tpu as pltpu
from jax.experimental.pallas import tpu_sc as plsc

N_NODES = 10000
N_PAD = 10240
E_EDGES = 160000
E_PAD = 163840
CHUNK = 64
NBUF = 5
LAG = NBUF - 2
ROWS_PER_TILE = E_PAD // CHUNK // 16
GROUPS = 10
GROUP_ROWS = ROWS_PER_TILE // GROUPS
EDGES_PER_TILE32 = E_PAD // 32
F_IN = 256
F_HALF = 128
F_OUT = 512
ROWB = 256


def _deg_kernel(dst_hbm, out_hbm, ldeg, dbuf):
    c = lax.axis_index("c")
    s = lax.axis_index("s")
    w = s * 2 + c
    zero16 = jnp.zeros((16,), jnp.float32)

    def zbody(i, carry):
        ldeg[pl.ds(i * 16, 16)] = zero16
        return carry

    lax.fori_loop(0, N_PAD // 16, zbody, 0)
    pltpu.sync_copy(
        dst_hbm.at[pl.ds(w * EDGES_PER_TILE32, EDGES_PER_TILE32)], dbuf)
    ones16 = jnp.ones((16,), jnp.float32)

    def body(i, carry):
        idx = dbuf[pl.ds(i * 16, 16)]
        plsc.addupdate_scatter(ldeg, [idx], ones16)
        return carry

    lax.fori_loop(0, EDGES_PER_TILE32 // 16, body, 0)
    pltpu.sync_copy(ldeg, out_hbm.at[pl.ds(w * N_PAD, N_PAD)])


def _norm_body(degp_ref, x_ref, xs_ref, dinv_ref):
    deg = jnp.sum(degp_ref[...], axis=0)
    dinv = jnp.where(deg > 0, lax.rsqrt(deg), 0.0)
    xr = x_ref[...] * dinv
    xs_ref[0] = xr[:, :F_HALF]
    xs_ref[1] = xr[:, F_HALF:]
    dinv_ref[...] = dinv


_norm_call = pl.pallas_call(
    _norm_body,
    grid=(N_PAD // ROWB,),
    in_specs=[
        pl.BlockSpec((32, ROWB, 1), lambda i: (0, i, 0)),
        pl.BlockSpec((ROWB, F_IN), lambda i: (i, 0)),
    ],
    out_specs=[
        pl.BlockSpec((2, ROWB, F_HALF), lambda i: (0, i, 0)),
        pl.BlockSpec((ROWB, 1), lambda i: (i, 0)),
    ],
    out_shape=[
        jax.ShapeDtypeStruct((2, N_PAD, F_HALF), jnp.float32),
        jax.ShapeDtypeStruct((N_PAD, 1), jnp.float32),
    ],
)


def _scatter_kernel(src2_hbm, dstr_hbm, xs_hbm, zeros_hbm, out_hbm,
                    agg_sh, srcb, dstb, rows, *sems):
    c = lax.axis_index("c")
    s = lax.axis_index("s")
    nrow = N_PAD // 16

    pltpu.sync_copy(zeros_hbm.at[pl.ds(s * nrow, nrow), :],
                    agg_sh.at[pl.ds(s * nrow, nrow), :])
    r0 = s * ROWS_PER_TILE
    plsc.subcore_barrier()

    gsem = sems[:NBUF]
    ssem = sems[NBUF:]
    for g in range(GROUPS):
        base = r0 + g * GROUP_ROWS
        pltpu.sync_copy(src2_hbm.at[c, pl.ds(base, GROUP_ROWS), :], srcb)
        pltpu.sync_copy(dstr_hbm.at[pl.ds(base, GROUP_ROWS), :], dstb)
        gd = [None] * NBUF
        sd = [None] * NBUF
        for j in range(GROUP_ROWS + LAG):
            if j < GROUP_ROWS:
                b = j % NBUF
                if j >= NBUF:
                    sd[b].wait()
                gd[b] = pltpu.async_copy(
                    xs_hbm.at[srcb.at[j]], rows.at[b], gsem[b])
            k = j - LAG
            if k >= 0:
                pb = k % NBUF
                gd[pb].wait()
                sd[pb] = pltpu.async_copy(
                    rows.at[pb], agg_sh.at[dstb.at[k]], ssem[pb], add=True)
        for i in range(min(NBUF, GROUP_ROWS)):
            sd[(GROUP_ROWS - 1 - i) % NBUF].wait()
    plsc.subcore_barrier()

    pltpu.sync_copy(agg_sh.at[pl.ds(s * nrow, nrow), :],
                    out_hbm.at[c, pl.ds(s * nrow, nrow), :])


def _linear_body(agg_ref, dinv_ref, w_ref, b_ref, out_ref):
    a = jnp.concatenate([agg_ref[0], agg_ref[1]], axis=1)
    acc = lax.dot_general(a, w_ref[...], (((1,), (1,)), ((), ())),
                          precision=lax.Precision.HIGHEST,
                          preferred_element_type=jnp.float32)
    out_ref[...] = jnp.maximum(acc * dinv_ref[...] + b_ref[...], 0.0)


_linear_call = pl.pallas_call(
    _linear_body,
    grid=(N_PAD // ROWB,),
    in_specs=[
        pl.BlockSpec((2, ROWB, F_HALF), lambda i: (0, i, 0)),
        pl.BlockSpec((ROWB, 1), lambda i: (i, 0)),
        pl.BlockSpec((F_OUT, F_IN), lambda i: (0, 0)),
        pl.BlockSpec((1, F_OUT), lambda i: (0, 0)),
    ],
    out_specs=pl.BlockSpec((ROWB, F_OUT), lambda i: (i, 0)),
    out_shape=jax.ShapeDtypeStruct((N_PAD, F_OUT), jnp.float32),
)


@functools.lru_cache(maxsize=1)
def _sc_kernels():
    mesh = plsc.VectorSubcoreMesh(core_axis_name="c", subcore_axis_name="s")
    params = pltpu.CompilerParams(needs_layout_passes=False,
                                  has_side_effects=True)
    deg = pl.kernel(
        _deg_kernel,
        out_type=jax.ShapeDtypeStruct((32 * N_PAD,), jnp.float32),
        mesh=mesh,
        compiler_params=params,
        scratch_types=[
            pltpu.VMEM((N_PAD,), jnp.float32),
            pltpu.VMEM((EDGES_PER_TILE32,), jnp.int32),
        ],
    )
    scat = pl.kernel(
        _scatter_kernel,
        out_type=jax.ShapeDtypeStruct((2, N_PAD, F_HALF), jnp.float32),
        mesh=mesh,
        compiler_params=params,
        scratch_types=[
            pltpu.VMEM_SHARED((N_PAD, F_HALF), jnp.float32),
            pltpu.VMEM((GROUP_ROWS, CHUNK), jnp.int32),
            pltpu.VMEM((GROUP_ROWS, CHUNK), jnp.int32),
            pltpu.VMEM((NBUF, CHUNK, F_HALF), jnp.float32),
        ] + [pltpu.SemaphoreType.DMA] * (2 * NBUF),
    )
    return deg, scat


def kernel(x, edge_index, W, b):
    deg_call, scatter_call = _sc_kernels()
    src = edge_index[0]
    dst = edge_index[1]
    pad_e = E_PAD - E_EDGES
    srcp = jnp.concatenate([src, jnp.zeros((pad_e,), jnp.int32)])
    dstp = jnp.concatenate([dst, jnp.full((pad_e,), N_NODES, jnp.int32)])
    src2 = jnp.stack([srcp, srcp + N_PAD]).reshape(2, E_PAD // CHUNK, CHUNK)
    dstr = dstp.reshape(E_PAD // CHUNK, CHUNK)

    deg_parts = deg_call(dstp)
    degp = deg_parts.reshape(32, N_PAD, 1)
    xp = jnp.pad(x, ((0, N_PAD - N_NODES), (0, 0)))

    xs3, dinv = _norm_call(degp, xp)
    xs_flat = xs3.reshape(2 * N_PAD, F_HALF)
    zeros_f = jnp.zeros((N_PAD, F_HALF), jnp.float32)
    agg3 = scatter_call(src2, dstr, xs_flat, zeros_f)

    out = _linear_call(agg3, dinv, W, b.reshape(1, F_OUT))
    return out[:N_NODES]


# speedup vs baseline: 2.3789x; 2.3789x over previous
import functools

import jax
import jax.numpy as jnp
from jax import lax
from jax.experimental import pallas as pl
from jax.experimental.pallas import tpu as pltpu
from jax.experimental.pallas import tpu_sc as plsc

N_NODES = 10000
N_PAD = 10240
E_EDGES = 160000
E_PAD = 163840
CHUNK = 64
NBUF = 5
LAG = NBUF - 2
ROWS_PER_TILE = E_PAD // CHUNK // 16
GROUPS = 10
GROUP_ROWS = ROWS_PER_TILE // GROUPS
EDGES_PER_TILE32 = E_PAD // 32
F_IN = 256
F_HALF = 128
F_OUT = 512
ROWB = 256


def _deg_kernel(dst_hbm, out_hbm, ldeg, dbuf):
    c = lax.axis_index("c")
    s = lax.axis_index("s")
    w = s * 2 + c
    zero16 = jnp.zeros((16,), jnp.float32)

    def zbody(i, carry):
        ldeg[pl.ds(i * 16, 16)] = zero16
        return carry

    lax.fori_loop(0, N_PAD // 16, zbody, 0)
    pltpu.sync_copy(
        dst_hbm.at[pl.ds(w * EDGES_PER_TILE32, EDGES_PER_TILE32)], dbuf)
    ones16 = jnp.ones((16,), jnp.float32)

    def body(i, carry):
        idx = dbuf[pl.ds(i * 16, 16)]
        plsc.addupdate_scatter(ldeg, [idx], ones16)
        return carry

    lax.fori_loop(0, EDGES_PER_TILE32 // 16, body, 0)
    pltpu.sync_copy(ldeg, out_hbm.at[pl.ds(w * N_PAD, N_PAD)])


def _norm_body(degp_ref, x_ref, xs_ref, dinv_ref):
    deg = lax.dot_general(degp_ref[...], jnp.ones((32, 1), jnp.float32),
                          (((0,), (0,)), ((), ())),
                          precision=lax.Precision.HIGHEST,
                          preferred_element_type=jnp.float32)
    dinv = jnp.where(deg > 0, lax.rsqrt(deg), 0.0)
    xr = x_ref[...] * dinv
    xs_ref[0] = xr[:, :F_HALF]
    xs_ref[1] = xr[:, F_HALF:]
    dinv_ref[...] = dinv


_norm_call = pl.pallas_call(
    _norm_body,
    grid=(N_PAD // ROWB,),
    in_specs=[
        pl.BlockSpec((32, ROWB), lambda i: (0, i)),
        pl.BlockSpec((ROWB, F_IN), lambda i: (i, 0)),
    ],
    out_specs=[
        pl.BlockSpec((2, ROWB, F_HALF), lambda i: (0, i, 0)),
        pl.BlockSpec((ROWB, 1), lambda i: (i, 0)),
    ],
    out_shape=[
        jax.ShapeDtypeStruct((2, N_PAD, F_HALF), jnp.float32),
        jax.ShapeDtypeStruct((N_PAD, 1), jnp.float32),
    ],
)


def _scatter_kernel(src2_hbm, dstr_hbm, xs_hbm, zeros_hbm, out_hbm,
                    agg_sh, srcb, dstb, rows, *sems):
    c = lax.axis_index("c")
    s = lax.axis_index("s")
    nrow = N_PAD // 16

    pltpu.sync_copy(zeros_hbm.at[pl.ds(s * nrow, nrow), :],
                    agg_sh.at[pl.ds(s * nrow, nrow), :])
    r0 = s * ROWS_PER_TILE
    plsc.subcore_barrier()

    gsem = sems[:NBUF]
    ssem = sems[NBUF:]
    for g in range(GROUPS):
        base = r0 + g * GROUP_ROWS
        pltpu.sync_copy(src2_hbm.at[c, pl.ds(base, GROUP_ROWS), :], srcb)
        pltpu.sync_copy(dstr_hbm.at[pl.ds(base, GROUP_ROWS), :], dstb)
        gd = [None] * NBUF
        sd = [None] * NBUF
        for j in range(GROUP_ROWS + LAG):
            if j < GROUP_ROWS:
                b = j % NBUF
                if j >= NBUF:
                    sd[b].wait()
                gd[b] = pltpu.async_copy(
                    xs_hbm.at[srcb.at[j]], rows.at[b], gsem[b])
            k = j - LAG
            if k >= 0:
                pb = k % NBUF
                gd[pb].wait()
                sd[pb] = pltpu.async_copy(
                    rows.at[pb], agg_sh.at[dstb.at[k]], ssem[pb], add=True)
        for i in range(min(NBUF, GROUP_ROWS)):
            sd[(GROUP_ROWS - 1 - i) % NBUF].wait()
    plsc.subcore_barrier()

    pltpu.sync_copy(agg_sh.at[pl.ds(s * nrow, nrow), :],
                    out_hbm.at[c, pl.ds(s * nrow, nrow), :])


def _linear_body(agg_ref, dinv_ref, w_ref, b_ref, out_ref):
    a = jnp.concatenate([agg_ref[0], agg_ref[1]], axis=1)
    acc = lax.dot_general(a, w_ref[...], (((1,), (1,)), ((), ())),
                          precision=lax.Precision.HIGHEST,
                          preferred_element_type=jnp.float32)
    out_ref[...] = jnp.maximum(acc * dinv_ref[...] + b_ref[...], 0.0)


_linear_call = pl.pallas_call(
    _linear_body,
    grid=(N_PAD // ROWB,),
    in_specs=[
        pl.BlockSpec((2, ROWB, F_HALF), lambda i: (0, i, 0)),
        pl.BlockSpec((ROWB, 1), lambda i: (i, 0)),
        pl.BlockSpec((F_OUT, F_IN), lambda i: (0, 0)),
        pl.BlockSpec((1, F_OUT), lambda i: (0, 0)),
    ],
    out_specs=pl.BlockSpec((ROWB, F_OUT), lambda i: (i, 0)),
    out_shape=jax.ShapeDtypeStruct((N_PAD, F_OUT), jnp.float32),
)


@functools.lru_cache(maxsize=1)
def _sc_kernels():
    mesh = plsc.VectorSubcoreMesh(core_axis_name="c", subcore_axis_name="s")
    params = pltpu.CompilerParams(needs_layout_passes=False)
    deg = pl.kernel(
        _deg_kernel,
        out_type=jax.ShapeDtypeStruct((32 * N_PAD,), jnp.float32),
        mesh=mesh,
        compiler_params=params,
        scratch_types=[
            pltpu.VMEM((N_PAD,), jnp.float32),
            pltpu.VMEM((EDGES_PER_TILE32,), jnp.int32),
        ],
    )
    scat = pl.kernel(
        _scatter_kernel,
        out_type=jax.ShapeDtypeStruct((2, N_PAD, F_HALF), jnp.float32),
        mesh=mesh,
        compiler_params=params,
        scratch_types=[
            pltpu.VMEM_SHARED((N_PAD, F_HALF), jnp.float32),
            pltpu.VMEM((GROUP_ROWS, CHUNK), jnp.int32),
            pltpu.VMEM((GROUP_ROWS, CHUNK), jnp.int32),
            pltpu.VMEM((NBUF, CHUNK, F_HALF), jnp.float32),
        ] + [pltpu.SemaphoreType.DMA] * (2 * NBUF),
    )
    return deg, scat


def kernel(x, edge_index, W, b):
    deg_call, scatter_call = _sc_kernels()
    src = edge_index[0]
    dst = edge_index[1]
    pad_e = E_PAD - E_EDGES
    pad_ids = N_NODES + (jnp.arange(pad_e, dtype=jnp.int32) % (N_PAD - N_NODES))
    srcp = jnp.concatenate([src, pad_ids])
    dstp = jnp.concatenate([dst, pad_ids])
    src2 = jnp.stack([srcp, srcp + N_PAD]).reshape(2, E_PAD // CHUNK, CHUNK)
    dstr = dstp.reshape(E_PAD // CHUNK, CHUNK)

    deg_parts = deg_call(dstp)
    degp = deg_parts.reshape(32, N_PAD)
    xp = jnp.pad(x, ((0, N_PAD - N_NODES), (0, 0)))

    xs3, dinv = _norm_call(degp, xp)
    xs_flat = xs3.reshape(2 * N_PAD, F_HALF)
    zeros_f = jnp.zeros((N_PAD, F_HALF), jnp.float32)
    agg3 = scatter_call(src2, dstr, xs_flat, zeros_f)

    out = _linear_call(agg3, dinv, W, b.reshape(1, F_OUT))
    return out[:N_NODES]


# speedup vs baseline: 2.4556x; 1.0322x over previous
import functools

import jax
import jax.numpy as jnp
from jax import lax
from jax.experimental import pallas as pl
from jax.experimental.pallas import tpu as pltpu
from jax.experimental.pallas import tpu_sc as plsc

N_NODES = 10000
N_PAD = 10240
E_EDGES = 160000
E_PAD = 163840
CHUNK = 64
NBUF = 5
LAG = NBUF - 2
ROWS_PER_TILE = E_PAD // CHUNK // 16
GROUPS = 10
GROUP_ROWS = ROWS_PER_TILE // GROUPS
EDGES_PER_TILE32 = E_PAD // 32
F_IN = 256
F_HALF = 128
F_OUT = 512
ROWB = 256


def _deg_kernel(dst_hbm, out_hbm, ldeg, dbuf):
    c = lax.axis_index("c")
    s = lax.axis_index("s")
    w = s * 2 + c
    zero16 = jnp.zeros((16,), jnp.float32)

    def zbody(i, carry):
        ldeg[pl.ds(i * 16, 16)] = zero16
        return carry

    lax.fori_loop(0, N_PAD // 16, zbody, 0)
    pltpu.sync_copy(
        dst_hbm.at[pl.ds(w * EDGES_PER_TILE32, EDGES_PER_TILE32)], dbuf)
    ones16 = jnp.ones((16,), jnp.float32)

    def body(i, carry):
        idx = dbuf[pl.ds(i * 16, 16)]
        plsc.addupdate_scatter(ldeg, [idx], ones16)
        return carry

    lax.fori_loop(0, EDGES_PER_TILE32 // 16, body, 0)
    pltpu.sync_copy(ldeg, out_hbm.at[pl.ds(w * N_PAD, N_PAD)])


def _norm_body(degp_ref, x_ref, xs_ref, dinv_ref):
    deg = lax.dot_general(degp_ref[...], jnp.ones((32, 1), jnp.float32),
                          (((0,), (0,)), ((), ())),
                          precision=lax.Precision.HIGHEST,
                          preferred_element_type=jnp.float32)
    dinv = jnp.where(deg > 0, lax.rsqrt(deg), 0.0)
    xr = x_ref[...] * dinv
    xs_ref[0] = xr[:, :F_HALF]
    xs_ref[1] = xr[:, F_HALF:]
    dinv_ref[...] = dinv


_norm_call = pl.pallas_call(
    _norm_body,
    grid=(N_PAD // ROWB,),
    in_specs=[
        pl.BlockSpec((32, ROWB), lambda i: (0, i)),
        pl.BlockSpec((ROWB, F_IN), lambda i: (i, 0)),
    ],
    out_specs=[
        pl.BlockSpec((2, ROWB, F_HALF), lambda i: (0, i, 0)),
        pl.BlockSpec((ROWB, 1), lambda i: (i, 0)),
    ],
    out_shape=[
        jax.ShapeDtypeStruct((2, N_PAD, F_HALF), jnp.float32),
        jax.ShapeDtypeStruct((N_PAD, 1), jnp.float32),
    ],
)


def _scatter_kernel(src2_hbm, dstr_hbm, xs_hbm, zeros_hbm, out_hbm,
                    agg_sh, srcb, dstb, rows, *sems):
    c = lax.axis_index("c")
    s = lax.axis_index("s")
    nrow = N_PAD // 16

    pltpu.sync_copy(zeros_hbm.at[pl.ds(s * nrow, nrow), :],
                    agg_sh.at[pl.ds(s * nrow, nrow), :])
    r0 = s * ROWS_PER_TILE
    plsc.subcore_barrier()

    gsem = sems[:NBUF]
    ssem = sems[NBUF:]
    for g in range(GROUPS):
        base = r0 + g * GROUP_ROWS
        pltpu.sync_copy(src2_hbm.at[c, pl.ds(base, GROUP_ROWS), :], srcb)
        pltpu.sync_copy(dstr_hbm.at[pl.ds(base, GROUP_ROWS), :], dstb)
        gd = [None] * NBUF
        sd = [None] * NBUF
        for j in range(GROUP_ROWS + LAG):
            if j < GROUP_ROWS:
                b = j % NBUF
                if j >= NBUF:
                    sd[b].wait()
                gd[b] = pltpu.async_copy(
                    xs_hbm.at[srcb.at[j]], rows.at[b], gsem[b])
            k = j - LAG
            if k >= 0:
                pb = k % NBUF
                gd[pb].wait()
                sd[pb] = pltpu.async_copy(
                    rows.at[pb], agg_sh.at[dstb.at[k]], ssem[pb], add=True)
        for i in range(min(NBUF, GROUP_ROWS)):
            sd[(GROUP_ROWS - 1 - i) % NBUF].wait()
    plsc.subcore_barrier()

    pltpu.sync_copy(agg_sh.at[pl.ds(s * nrow, nrow), :],
                    out_hbm.at[c, pl.ds(s * nrow, nrow), :])


def _linear_body(agg_ref, dinv_ref, w_ref, b_ref, out_ref):
    a = jnp.concatenate([agg_ref[0], agg_ref[1]], axis=1)
    acc = lax.dot_general(a, w_ref[...], (((1,), (1,)), ((), ())),
                          preferred_element_type=jnp.float32)
    out_ref[...] = jnp.maximum(acc * dinv_ref[...] + b_ref[...], 0.0)


_linear_call = pl.pallas_call(
    _linear_body,
    grid=(N_PAD // ROWB,),
    in_specs=[
        pl.BlockSpec((2, ROWB, F_HALF), lambda i: (0, i, 0)),
        pl.BlockSpec((ROWB, 1), lambda i: (i, 0)),
        pl.BlockSpec((F_OUT, F_IN), lambda i: (0, 0)),
        pl.BlockSpec((1, F_OUT), lambda i: (0, 0)),
    ],
    out_specs=pl.BlockSpec((ROWB, F_OUT), lambda i: (i, 0)),
    out_shape=jax.ShapeDtypeStruct((N_PAD, F_OUT), jnp.float32),
)


@functools.lru_cache(maxsize=1)
def _sc_kernels():
    mesh = plsc.VectorSubcoreMesh(core_axis_name="c", subcore_axis_name="s")
    params = pltpu.CompilerParams(needs_layout_passes=False)
    deg = pl.kernel(
        _deg_kernel,
        out_type=jax.ShapeDtypeStruct((32 * N_PAD,), jnp.float32),
        mesh=mesh,
        compiler_params=params,
        scratch_types=[
            pltpu.VMEM((N_PAD,), jnp.float32),
            pltpu.VMEM((EDGES_PER_TILE32,), jnp.int32),
        ],
    )
    scat = pl.kernel(
        _scatter_kernel,
        out_type=jax.ShapeDtypeStruct((2, N_PAD, F_HALF), jnp.float32),
        mesh=mesh,
        compiler_params=params,
        scratch_types=[
            pltpu.VMEM_SHARED((N_PAD, F_HALF), jnp.float32),
            pltpu.VMEM((GROUP_ROWS, CHUNK), jnp.int32),
            pltpu.VMEM((GROUP_ROWS, CHUNK), jnp.int32),
            pltpu.VMEM((NBUF, CHUNK, F_HALF), jnp.float32),
        ] + [pltpu.SemaphoreType.DMA] * (2 * NBUF),
    )
    return deg, scat


def kernel(x, edge_index, W, b):
    deg_call, scatter_call = _sc_kernels()
    src = edge_index[0]
    dst = edge_index[1]
    pad_e = E_PAD - E_EDGES
    pad_ids = N_NODES + (jnp.arange(pad_e, dtype=jnp.int32) % (N_PAD - N_NODES))
    srcp = jnp.concatenate([src, pad_ids])
    dstp = jnp.concatenate([dst, pad_ids])
    src2 = jnp.stack([srcp, srcp + N_PAD]).reshape(2, E_PAD // CHUNK, CHUNK)
    dstr = dstp.reshape(E_PAD // CHUNK, CHUNK)

    deg_parts = deg_call(dstp)
    degp = deg_parts.reshape(32, N_PAD)
    xp = jnp.pad(x, ((0, N_PAD - N_NODES), (0, 0)))

    xs3, dinv = _norm_call(degp, xp)
    xs_flat = xs3.reshape(2 * N_PAD, F_HALF)
    zeros_f = jnp.zeros((N_PAD, F_HALF), jnp.float32)
    agg3 = scatter_call(src2, dstr, xs_flat, zeros_f)

    out = _linear_call(agg3, dinv, W, b.reshape(1, F_OUT))
    return out[:N_NODES]


# speedup vs baseline: 2.6079x; 1.0620x over previous
import functools

import jax
import jax.numpy as jnp
from jax import lax
from jax.experimental import pallas as pl
from jax.experimental.pallas import tpu as pltpu
from jax.experimental.pallas import tpu_sc as plsc

N_NODES = 10000
N_PAD = 10240
E_EDGES = 160000
E_PAD = 163840
CHUNK = 64
NBUF = 5
LAG = NBUF - 2
ROWS_PER_TILE = E_PAD // CHUNK // 16
GROUPS = 10
GROUP_ROWS = ROWS_PER_TILE // GROUPS
EDGES_PER_TILE32 = E_PAD // 32
F_IN = 256
F_HALF = 128
F_OUT = 512
ROWB = 256


def _deg_kernel(dst_hbm, out_hbm, ldeg, dbuf):
    c = lax.axis_index("c")
    s = lax.axis_index("s")
    w = s * 2 + c
    zero16 = jnp.zeros((16,), jnp.float32)

    def zbody(i, carry):
        ldeg[pl.ds(i * 16, 16)] = zero16
        return carry

    lax.fori_loop(0, N_PAD // 16, zbody, 0)
    pltpu.sync_copy(
        dst_hbm.at[pl.ds(w * EDGES_PER_TILE32, EDGES_PER_TILE32)], dbuf)
    ones16 = jnp.ones((16,), jnp.float32)

    def body(i, carry):
        idx = dbuf[pl.ds(i * 16, 16)]
        plsc.addupdate_scatter(ldeg, [idx], ones16)
        return carry

    lax.fori_loop(0, EDGES_PER_TILE32 // 16, body, 0)
    pltpu.sync_copy(ldeg, out_hbm.at[pl.ds(w * N_PAD, N_PAD)])


def _norm_body(degp_ref, x_ref, xs_ref, dinv_ref):
    deg = lax.dot_general(degp_ref[...], jnp.ones((32, 1), jnp.float32),
                          (((0,), (0,)), ((), ())),
                          precision=lax.Precision.HIGHEST,
                          preferred_element_type=jnp.float32)
    dinv = jnp.where(deg > 0, lax.rsqrt(deg), 0.0)
    xr = x_ref[...] * dinv
    xs_ref[0] = xr[:, :F_HALF]
    xs_ref[1] = xr[:, F_HALF:]
    dinv_ref[...] = dinv


_norm_call = pl.pallas_call(
    _norm_body,
    grid=(N_PAD // ROWB,),
    in_specs=[
        pl.BlockSpec((32, ROWB), lambda i: (0, i)),
        pl.BlockSpec((ROWB, F_IN), lambda i: (i, 0)),
    ],
    out_specs=[
        pl.BlockSpec((2, ROWB, F_HALF), lambda i: (0, i, 0)),
        pl.BlockSpec((ROWB, 1), lambda i: (i, 0)),
    ],
    out_shape=[
        jax.ShapeDtypeStruct((2, N_PAD, F_HALF), jnp.float32),
        jax.ShapeDtypeStruct((N_PAD, 1), jnp.float32),
    ],
)


def _scatter_kernel(src2_hbm, dstr_hbm, xs_hbm, zeros_hbm, out_hbm,
                    agg_sh, srcb, dstb, rows, *sems):
    c = lax.axis_index("c")
    s = lax.axis_index("s")
    nrow = N_PAD // 16

    pltpu.sync_copy(zeros_hbm.at[pl.ds(s * nrow, nrow), :],
                    agg_sh.at[pl.ds(s * nrow, nrow), :])
    r0 = s * ROWS_PER_TILE
    plsc.subcore_barrier()

    gsem = sems[:NBUF]
    ssem = sems[NBUF:]
    for g in range(GROUPS):
        base = r0 + g * GROUP_ROWS
        pltpu.sync_copy(src2_hbm.at[c, pl.ds(base, GROUP_ROWS), :], srcb)
        pltpu.sync_copy(dstr_hbm.at[pl.ds(base, GROUP_ROWS), :], dstb)
        gd = [None] * NBUF
        sd = [None] * NBUF
        for j in range(GROUP_ROWS + LAG):
            if j < GROUP_ROWS:
                b = j % NBUF
                if j >= NBUF:
                    sd[b].wait()
                gd[b] = pltpu.async_copy(
                    xs_hbm.at[srcb.at[j]], rows.at[b], gsem[b])
            k = j - LAG
            if k >= 0:
                pb = k % NBUF
                gd[pb].wait()
                sd[pb] = pltpu.async_copy(
                    rows.at[pb], agg_sh.at[dstb.at[k]], ssem[pb], add=True)
        for i in range(min(NBUF, GROUP_ROWS)):
            sd[(GROUP_ROWS - 1 - i) % NBUF].wait()
    plsc.subcore_barrier()

    pltpu.sync_copy(agg_sh.at[pl.ds(s * nrow, nrow), :],
                    out_hbm.at[c, pl.ds(s * nrow, nrow), :])


def _linear_body(agg_ref, dinv_ref, w_ref, b_ref, out_ref):
    a = jnp.concatenate([agg_ref[0], agg_ref[1]], axis=1)
    acc = lax.dot_general(a, w_ref[...], (((1,), (1,)), ((), ())),
                          preferred_element_type=jnp.float32)
    out_ref[...] = jnp.maximum(acc * dinv_ref[...] + b_ref[...], 0.0)


_linear_call = pl.pallas_call(
    _linear_body,
    grid=(N_PAD // ROWB,),
    in_specs=[
        pl.BlockSpec((2, ROWB, F_HALF), lambda i: (0, i, 0)),
        pl.BlockSpec((ROWB, 1), lambda i: (i, 0)),
        pl.BlockSpec((F_OUT, F_IN), lambda i: (0, 0)),
        pl.BlockSpec((1, F_OUT), lambda i: (0, 0)),
    ],
    out_specs=pl.BlockSpec((ROWB, F_OUT), lambda i: (i, 0)),
    out_shape=jax.ShapeDtypeStruct((N_NODES, F_OUT), jnp.float32),
)


@functools.lru_cache(maxsize=1)
def _sc_kernels():
    mesh = plsc.VectorSubcoreMesh(core_axis_name="c", subcore_axis_name="s")
    params = pltpu.CompilerParams(needs_layout_passes=False)
    deg = pl.kernel(
        _deg_kernel,
        out_type=jax.ShapeDtypeStruct((32 * N_PAD,), jnp.float32),
        mesh=mesh,
        compiler_params=params,
        scratch_types=[
            pltpu.VMEM((N_PAD,), jnp.float32),
            pltpu.VMEM((EDGES_PER_TILE32,), jnp.int32),
        ],
    )
    scat = pl.kernel(
        _scatter_kernel,
        out_type=jax.ShapeDtypeStruct((2, N_PAD, F_HALF), jnp.float32),
        mesh=mesh,
        compiler_params=params,
        scratch_types=[
            pltpu.VMEM_SHARED((N_PAD, F_HALF), jnp.float32),
            pltpu.VMEM((GROUP_ROWS, CHUNK), jnp.int32),
            pltpu.VMEM((GROUP_ROWS, CHUNK), jnp.int32),
            pltpu.VMEM((NBUF, CHUNK, F_HALF), jnp.float32),
        ] + [pltpu.SemaphoreType.DMA] * (2 * NBUF),
    )
    return deg, scat


def kernel(x, edge_index, W, b):
    deg_call, scatter_call = _sc_kernels()
    src = edge_index[0]
    dst = edge_index[1]
    pad_e = E_PAD - E_EDGES
    pad_ids = N_NODES + (jnp.arange(pad_e, dtype=jnp.int32) % (N_PAD - N_NODES))
    srcp = jnp.concatenate([src, pad_ids])
    dstp = jnp.concatenate([dst, pad_ids])
    src2 = jnp.stack([srcp, srcp + N_PAD]).reshape(2, E_PAD // CHUNK, CHUNK)
    dstr = dstp.reshape(E_PAD // CHUNK, CHUNK)

    deg_parts = deg_call(dstp)
    degp = deg_parts.reshape(32, N_PAD)
    xp = jnp.pad(x, ((0, N_PAD - N_NODES), (0, 0)))

    xs3, dinv = _norm_call(degp, xp)
    xs_flat = xs3.reshape(2 * N_PAD, F_HALF)
    zeros_f = jnp.zeros((N_PAD, F_HALF), jnp.float32)
    agg3 = scatter_call(src2, dstr, xs_flat, zeros_f)

    return _linear_call(agg3, dinv, W, b.reshape(1, F_OUT))
